# baseline (device time: 9657 ns/iter reference)
import jax
import jax.numpy as jnp
from jax import lax
from jax.experimental import pallas as pl
from jax.experimental.pallas import tpu as pltpu

_QC = 2


def kernel(x):
    m, n = x.shape
    half = n // 2
    qr = m // _QC

    def body(x_ref, out_ref, qbuf, sbuf, qrecv, srecv, lbuf, dbuf,
             q_send_sems, q_recv_sems, s_send_sem, s_recv_sem,
             lout_sem, dout_sems):
        my_x = lax.axis_index("x")
        my_y = lax.axis_index("y")
        my_z = lax.axis_index("z")

        def exchange(yv):
            peer = 1 - yv

            xb = x_ref[:, peer * half:(peer + 1) * half]
            absmax = jnp.max(jnp.abs(xb), axis=0, keepdims=True)
            absmax = jnp.where(absmax > 0.0, absmax, 1.0)
            qbuf[...] = jnp.round(xb * (127.0 / absmax)).astype(jnp.int8)
            sbuf[...] = absmax * (1.0 / 127.0)
            lbuf[...] = x_ref[:, yv * half:(yv + 1) * half].astype(
                jnp.bfloat16
            )

            barrier_sem = pltpu.get_barrier_semaphore()
            pl.semaphore_signal(
                barrier_sem,
                inc=1,
                device_id=(my_x, peer, my_z),
                device_id_type=pl.DeviceIdType.MESH,
            )
            pl.semaphore_wait(barrier_sem, 1)

            rdma_s = pltpu.make_async_remote_copy(
                src_ref=sbuf,
                dst_ref=srecv,
                send_sem=s_send_sem,
                recv_sem=s_recv_sem,
                device_id=(my_x, peer, my_z),
                device_id_type=pl.DeviceIdType.MESH,
            )
            rdma_s.start()
            rdma_qs = []
            for k in range(_QC):
                r = pltpu.make_async_remote_copy(
                    src_ref=qbuf.at[pl.ds(k * qr, qr)],
                    dst_ref=qrecv.at[pl.ds(k * qr, qr)],
                    send_sem=q_send_sems.at[k],
                    recv_sem=q_recv_sems.at[k],
                    device_id=(my_x, peer, my_z),
                    device_id_type=pl.DeviceIdType.MESH,
                )
                r.start()
                rdma_qs.append(r)

            lout = pltpu.make_async_copy(
                lbuf, out_ref.at[pl.ds(yv * m, m)], lout_sem
            )
            lout.start()

            rdma_s.wait()
            douts = []
            for k in range(_QC):
                rdma_qs[k].wait()
                dbuf[k * qr:(k + 1) * qr, :] = (
                    qrecv[k * qr:(k + 1) * qr, :].astype(jnp.float32)
                    * srecv[...]
                ).astype(jnp.bfloat16)
                d = pltpu.make_async_copy(
                    dbuf.at[pl.ds(k * qr, qr)],
                    out_ref.at[pl.ds(peer * m + k * qr, qr)],
                    dout_sems.at[k],
                )
                d.start()
                douts.append(d)

            lout.wait()
            for d in douts:
                d.wait()

        @pl.when(my_y == 0)
        def _():
            exchange(0)

        @pl.when(my_y == 1)
        def _():
            exchange(1)

    return pl.pallas_call(
        body,
        out_shape=jax.ShapeDtypeStruct((2 * m, half), jnp.bfloat16),
        in_specs=[pl.BlockSpec(memory_space=pltpu.VMEM)],
        out_specs=pl.BlockSpec(memory_space=pltpu.MemorySpace.HBM),
        scratch_shapes=[
            pltpu.VMEM((m, half), jnp.int8),
            pltpu.VMEM((1, half), jnp.float32),
            pltpu.VMEM((m, half), jnp.int8),
            pltpu.VMEM((1, half), jnp.float32),
            pltpu.VMEM((m, half), jnp.bfloat16),
            pltpu.VMEM((m, half), jnp.bfloat16),
            pltpu.SemaphoreType.DMA((_QC,)),
            pltpu.SemaphoreType.DMA((_QC,)),
            pltpu.SemaphoreType.DMA,
            pltpu.SemaphoreType.DMA,
            pltpu.SemaphoreType.DMA,
            pltpu.SemaphoreType.DMA((_QC,)),
        ],
        compiler_params=pltpu.CompilerParams(collective_id=0),
    )(x)


# device time: 9594 ns/iter; 1.0066x vs baseline; 1.0066x over previous
import jax
import jax.numpy as jnp
from jax import lax
from jax.experimental import pallas as pl
from jax.experimental.pallas import tpu as pltpu


def kernel(x):
    m, n = x.shape
    half = n // 2

    def body(x_ref, out_ref, qbuf, sbuf, qrecv, srecv,
             q_send_sem, q_recv_sem, s_send_sem, s_recv_sem):
        my_x = lax.axis_index("x")
        my_y = lax.axis_index("y")
        my_z = lax.axis_index("z")

        def exchange(yv):
            peer = 1 - yv

            xb = x_ref[:, peer * half:(peer + 1) * half]
            absmax = jnp.max(jnp.abs(xb), axis=0, keepdims=True)
            absmax = jnp.where(absmax > 0.0, absmax, 1.0)
            qbuf[...] = jnp.round(xb * (127.0 / absmax)).astype(jnp.int8)
            sbuf[...] = absmax * (1.0 / 127.0)

            barrier_sem = pltpu.get_barrier_semaphore()
            pl.semaphore_signal(
                barrier_sem,
                inc=1,
                device_id=(my_x, peer, my_z),
                device_id_type=pl.DeviceIdType.MESH,
            )
            pl.semaphore_wait(barrier_sem, 1)

            rdma_s = pltpu.make_async_remote_copy(
                src_ref=sbuf,
                dst_ref=srecv,
                send_sem=s_send_sem,
                recv_sem=s_recv_sem,
                device_id=(my_x, peer, my_z),
                device_id_type=pl.DeviceIdType.MESH,
            )
            rdma_s.start()
            rdma_q = pltpu.make_async_remote_copy(
                src_ref=qbuf,
                dst_ref=qrecv,
                send_sem=q_send_sem,
                recv_sem=q_recv_sem,
                device_id=(my_x, peer, my_z),
                device_id_type=pl.DeviceIdType.MESH,
            )
            rdma_q.start()

            out_ref[yv * m:(yv + 1) * m, :] = x_ref[
                :, yv * half:(yv + 1) * half
            ].astype(jnp.bfloat16)

            rdma_s.wait()
            rdma_q.wait()
            out_ref[peer * m:(peer + 1) * m, :] = (
                qrecv[...].astype(jnp.float32) * srecv[...]
            ).astype(jnp.bfloat16)

        @pl.when(my_y == 0)
        def _():
            exchange(0)

        @pl.when(my_y == 1)
        def _():
            exchange(1)

    return pl.pallas_call(
        body,
        out_shape=jax.ShapeDtypeStruct((2 * m, half), jnp.bfloat16),
        in_specs=[pl.BlockSpec(memory_space=pltpu.VMEM)],
        out_specs=pl.BlockSpec(memory_space=pltpu.VMEM),
        scratch_shapes=[
            pltpu.VMEM((m, half), jnp.int8),
            pltpu.VMEM((1, half), jnp.float32),
            pltpu.VMEM((m, half), jnp.int8),
            pltpu.VMEM((1, half), jnp.float32),
            pltpu.SemaphoreType.DMA,
            pltpu.SemaphoreType.DMA,
            pltpu.SemaphoreType.DMA,
            pltpu.SemaphoreType.DMA,
        ],
        compiler_params=pltpu.CompilerParams(collective_id=0),
    )(x)
